# baseline (speedup 1.0000x reference)
import functools

import jax
import jax.numpy as jnp
from jax import lax
from jax.experimental import pallas as pl
from jax.experimental.pallas import tpu as pltpu
from jax.experimental.pallas import tpu_sc as plsc

H = 224
W = 224
P = H * W
K = 9
C = 96
N = K * P
NW = 32
NP = N // NW
CH = 112
NCHUNK = NP // CH
NV = C // 16


def _transpose_in(inp2d):
    PB = 512

    def body(x_ref, o_ref):
        o_ref[...] = x_ref[...].T

    return pl.pallas_call(
        body,
        grid=(P // PB,),
        in_specs=[pl.BlockSpec((C, PB), lambda j: (0, j))],
        out_specs=pl.BlockSpec((PB, C), lambda j: (j, 0)),
        out_shape=jax.ShapeDtypeStruct((P, C), jnp.float32),
    )(inp2d)


def _prep(off):
    R = 56

    def body(o_ref, idx_ref, wgt_ref):
        k = pl.program_id(0)
        r = pl.program_id(1)
        ki = (k // 3).astype(jnp.float32)
        kj = (k % 3).astype(jnp.float32)
        ho = lax.broadcasted_iota(jnp.int32, (R, W), 0) + r * R
        wo = lax.broadcasted_iota(jnp.int32, (R, W), 1)
        y = (ho - 1).astype(jnp.float32) + ki + o_ref[0]
        x = (wo - 1).astype(jnp.float32) + kj + o_ref[1]
        y0 = jnp.floor(y)
        x0 = jnp.floor(x)
        ly = y - y0
        lx = x - x0
        hy = 1.0 - ly
        hx = 1.0 - lx
        y1 = y0 + 1.0
        x1 = x0 + 1.0
        corners = ((y0, x0, hy, hx), (y0, x1, hy, lx),
                   (y1, x0, ly, hx), (y1, x1, ly, lx))
        for ci, (yf, xf, wy, wx) in enumerate(corners):
            valid = ((yf >= 0.0) & (yf <= float(H - 1))
                     & (xf >= 0.0) & (xf <= float(W - 1)))
            yc = jnp.clip(yf, 0.0, float(H - 1))
            xc = jnp.clip(xf, 0.0, float(W - 1))
            idx_ref[ci, 0] = (yc * float(W) + xc).astype(jnp.int32)
            wgt_ref[ci, 0] = wy * wx * valid.astype(jnp.float32)

    return pl.pallas_call(
        body,
        grid=(K, H // R),
        in_specs=[pl.BlockSpec((2, R, W), lambda k, r: (k, r, 0))],
        out_specs=[
            pl.BlockSpec((4, 1, R, W), lambda k, r: (0, k, r, 0)),
            pl.BlockSpec((4, 1, R, W), lambda k, r: (0, k, r, 0)),
        ],
        out_shape=[
            jax.ShapeDtypeStruct((4, K, H, W), jnp.int32),
            jax.ShapeDtypeStruct((4, K, H, W), jnp.float32),
        ],
    )(off)


def _sc_gather_blend(table, idx, wgt):
    mesh = plsc.VectorSubcoreMesh(core_axis_name="c", subcore_axis_name="s")

    @functools.partial(
        pl.kernel,
        out_type=jax.ShapeDtypeStruct((C * K, P), jnp.float32),
        mesh=mesh,
        scratch_types=[
            pltpu.VMEM((2, 4, CH), jnp.int32),
            pltpu.VMEM((2, 4, CH), jnp.float32),
            pltpu.VMEM((2, 4 * CH, C), jnp.float32),
            pltpu.VMEM((2, C, CH), jnp.float32),
            pltpu.SemaphoreType.DMA,
            pltpu.SemaphoreType.DMA,
            pltpu.SemaphoreType.DMA,
            pltpu.SemaphoreType.DMA,
        ],
        compiler_params=pltpu.CompilerParams(use_tc_tiling_on_sc=False,
                                             needs_layout_passes=False),
    )
    def run(table_hbm, idx_hbm, wgt_hbm, o_hbm, idx_v, wgt_v, rows_v, out_t,
            gs0, gs1, os0, os1):
        gsem = (gs0, gs1)
        osem = (os0, os1)
        wid = lax.axis_index("s") * 2 + lax.axis_index("c")
        base = wid * NP

        def fetch(ci, b):
            q0 = base + ci * CH
            pltpu.sync_copy(idx_hbm.at[:, pl.ds(q0, CH)], idx_v.at[b])
            pltpu.sync_copy(wgt_hbm.at[:, pl.ds(q0, CH)], wgt_v.at[b])
            for j in range(4):
                pltpu.async_copy(table_hbm.at[idx_v.at[b, j]],
                                 rows_v.at[b, pl.ds(j * CH, CH)], gsem[b])

        def drain_gathers(b):
            for j in range(4):
                pltpu.make_async_copy(table_hbm.at[pl.ds(0, CH)],
                                      rows_v.at[b, pl.ds(j * CH, CH)],
                                      gsem[b]).wait()

        def drain_out(b):
            pltpu.make_async_copy(o_hbm.at[pl.ds(0, C), pl.ds(0, CH)],
                                  out_t.at[b], osem[b]).wait()

        def blend(b):
            iot = lax.iota(jnp.int32, 16)
            for g in range(CH // 16):
                r0 = g * 16
                ridx = [iot + (j * CH + r0) for j in range(4)]
                wv = [wgt_v[b, j, pl.ds(r0, 16)] for j in range(4)]

                def c_body(c, carry, ridx=ridx, wv=wv, r0=r0):
                    cidx = jnp.full((16,), 0, jnp.int32) + c
                    acc = plsc.load_gather(rows_v.at[b], [ridx[0], cidx]) * wv[0]
                    acc = acc + plsc.load_gather(rows_v.at[b], [ridx[1], cidx]) * wv[1]
                    acc = acc + plsc.load_gather(rows_v.at[b], [ridx[2], cidx]) * wv[2]
                    acc = acc + plsc.load_gather(rows_v.at[b], [ridx[3], cidx]) * wv[3]
                    out_t[b, c, pl.ds(r0, 16)] = acc
                    return carry

                lax.fori_loop(0, C, c_body, 0)

        def fire_out(ci, b):
            q0 = base + ci * CH
            k = q0 // P
            p0 = q0 - k * P
            for c in range(C):
                pltpu.async_copy(out_t.at[b, c],
                                 o_hbm.at[c * K + k, pl.ds(p0, CH)], osem[b])

        fetch(0, 0)

        def pair_body(h, carry):
            for b in range(2):
                ci = 2 * h + b

                @pl.when(ci + 1 < NCHUNK)
                def _():
                    fetch(ci + 1, b ^ 1)

                drain_gathers(b)

                @pl.when(ci >= 2)
                def _():
                    drain_out(b)

                blend(b)
                fire_out(ci, b)
            return carry

        lax.fori_loop(0, NCHUNK // 2, pair_body, 0)
        drain_out(0)
        drain_out(1)

    return run(table, idx, wgt)


def kernel(input, offset):
    inp2d = input.reshape(C, P)
    off = offset.reshape(2 * K, H, W)
    table = _transpose_in(inp2d)
    idx, wgt = _prep(off)
    out = _sc_gather_blend(table, idx.reshape(4, N), wgt.reshape(4, N))
    return out.reshape(1, C * K, P)


# speedup vs baseline: 3.1721x; 3.1721x over previous
import functools

import jax
import jax.numpy as jnp
from jax import lax
from jax.experimental import pallas as pl
from jax.experimental.pallas import tpu as pltpu
from jax.experimental.pallas import tpu_sc as plsc

H = 224
W = 224
P = H * W
K = 9
C = 96
N = K * P
NW = 32
NP = N // NW
CH = 112
NCHUNK = NP // CH
NV = C // 16
OCH = 113


def _transpose_in(inp2d):
    PB = 512

    def body(x_ref, o_ref):
        o_ref[...] = x_ref[...].T

    return pl.pallas_call(
        body,
        grid=(P // PB,),
        in_specs=[pl.BlockSpec((C, PB), lambda j: (0, j))],
        out_specs=pl.BlockSpec((PB, C), lambda j: (j, 0)),
        out_shape=jax.ShapeDtypeStruct((P, C), jnp.float32),
    )(inp2d)


def _prep(off):
    R = 56

    def body(o_ref, idx_ref, wgt_ref):
        k = pl.program_id(0)
        r = pl.program_id(1)
        ki = (k // 3).astype(jnp.float32)
        kj = (k % 3).astype(jnp.float32)
        ho = lax.broadcasted_iota(jnp.int32, (R, W), 0) + r * R
        wo = lax.broadcasted_iota(jnp.int32, (R, W), 1)
        y = (ho - 1).astype(jnp.float32) + ki + o_ref[0]
        x = (wo - 1).astype(jnp.float32) + kj + o_ref[1]
        y0 = jnp.floor(y)
        x0 = jnp.floor(x)
        ly = y - y0
        lx = x - x0
        hy = 1.0 - ly
        hx = 1.0 - lx
        y1 = y0 + 1.0
        x1 = x0 + 1.0
        corners = ((y0, x0, hy, hx), (y0, x1, hy, lx),
                   (y1, x0, ly, hx), (y1, x1, ly, lx))
        for ci, (yf, xf, wy, wx) in enumerate(corners):
            valid = ((yf >= 0.0) & (yf <= float(H - 1))
                     & (xf >= 0.0) & (xf <= float(W - 1)))
            yc = jnp.clip(yf, 0.0, float(H - 1))
            xc = jnp.clip(xf, 0.0, float(W - 1))
            idx_ref[ci, 0] = (yc * float(W) + xc).astype(jnp.int32)
            wgt_ref[ci, 0] = wy * wx * valid.astype(jnp.float32)

    return pl.pallas_call(
        body,
        grid=(K, H // R),
        in_specs=[pl.BlockSpec((2, R, W), lambda k, r: (k, r, 0))],
        out_specs=[
            pl.BlockSpec((4, 1, R, W), lambda k, r: (0, k, r, 0)),
            pl.BlockSpec((4, 1, R, W), lambda k, r: (0, k, r, 0)),
        ],
        out_shape=[
            jax.ShapeDtypeStruct((4, K, H, W), jnp.int32),
            jax.ShapeDtypeStruct((4, K, H, W), jnp.float32),
        ],
    )(off)


def _sc_gather_blend(table, idx, wgt):
    mesh = plsc.VectorSubcoreMesh(core_axis_name="c", subcore_axis_name="s")

    @functools.partial(
        pl.kernel,
        out_type=jax.ShapeDtypeStruct((C * K, P), jnp.float32),
        mesh=mesh,
        scratch_types=[
            pltpu.VMEM((2, 4, CH), jnp.int32),
            pltpu.VMEM((2, 4, CH), jnp.float32),
            pltpu.VMEM((2, 4 * CH, C), jnp.float32),
            pltpu.VMEM((2, C, OCH), jnp.float32),
            pltpu.SemaphoreType.DMA,
            pltpu.SemaphoreType.DMA,
            pltpu.SemaphoreType.DMA,
            pltpu.SemaphoreType.DMA,
        ],
        compiler_params=pltpu.CompilerParams(use_tc_tiling_on_sc=False,
                                             needs_layout_passes=False),
    )
    def run(table_hbm, idx_hbm, wgt_hbm, o_hbm, idx_v, wgt_v, rows_v, out_t,
            gs0, gs1, os0, os1):
        gsem = (gs0, gs1)
        osem = (os0, os1)
        wid = lax.axis_index("s") * 2 + lax.axis_index("c")
        base = wid * NP

        def fetch(ci, b):
            q0 = base + ci * CH
            pltpu.sync_copy(idx_hbm.at[:, pl.ds(q0, CH)], idx_v.at[b])
            pltpu.sync_copy(wgt_hbm.at[:, pl.ds(q0, CH)], wgt_v.at[b])
            for j in range(4):
                pltpu.async_copy(table_hbm.at[idx_v.at[b, j]],
                                 rows_v.at[b, pl.ds(j * CH, CH)], gsem[b])

        def drain_gathers(b):
            for j in range(4):
                pltpu.make_async_copy(table_hbm.at[pl.ds(0, CH)],
                                      rows_v.at[b, pl.ds(j * CH, CH)],
                                      gsem[b]).wait()

        def drain_out(b):
            pltpu.make_async_copy(o_hbm.at[pl.ds(0, C), pl.ds(0, CH)],
                                  out_t.at[b, :, pl.ds(0, CH)], osem[b]).wait()

        def blend(b):
            iot = lax.iota(jnp.int32, 16)
            cidx = [iot + v * 16 for v in range(NV)]

            def g_body(g, carry):
                r0 = g * 16
                wv = [wgt_v[b, j, pl.ds(r0, 16)] for j in range(4)]
                for e in range(16):
                    r = r0 + e
                    ridx = jnp.full((16,), 0, jnp.int32) + r
                    for v in range(NV):
                        sl = pl.ds(v * 16, 16)
                        acc = rows_v[b, r, sl] * wv[0][e]
                        acc = acc + rows_v[b, CH + r, sl] * wv[1][e]
                        acc = acc + rows_v[b, 2 * CH + r, sl] * wv[2][e]
                        acc = acc + rows_v[b, 3 * CH + r, sl] * wv[3][e]
                        plsc.store_scatter(out_t.at[b], [cidx[v], ridx], acc)
                return carry

            lax.fori_loop(0, CH // 16, g_body, 0)

        def fire_out(ci, b):
            q0 = base + ci * CH
            k = q0 // P
            p0 = q0 - k * P
            for c in range(C):
                pltpu.async_copy(out_t.at[b, c, pl.ds(0, CH)],
                                 o_hbm.at[c * K + k, pl.ds(p0, CH)], osem[b])

        fetch(0, 0)

        def pair_body(h, carry):
            for b in range(2):
                ci = 2 * h + b

                @pl.when(ci + 1 < NCHUNK)
                def _():
                    fetch(ci + 1, b ^ 1)

                drain_gathers(b)

                @pl.when(ci >= 2)
                def _():
                    drain_out(b)

                blend(b)
                fire_out(ci, b)
            return carry

        lax.fori_loop(0, NCHUNK // 2, pair_body, 0)
        drain_out(0)
        drain_out(1)

    return run(table, idx, wgt)


def kernel(input, offset):
    inp2d = input.reshape(C, P)
    off = offset.reshape(2 * K, H, W)
    table = _transpose_in(inp2d)
    idx, wgt = _prep(off)
    out = _sc_gather_blend(table, idx.reshape(4, N), wgt.reshape(4, N))
    return out.reshape(1, C * K, P)
